# Initial kernel scaffold; baseline (speedup 1.0000x reference)
import jax
import jax.numpy as jnp
from jax.experimental import pallas as pl


def kernel(n_sample, rays_directions, rays_origins):
    raise NotImplementedError("write your pallas kernel here")


# speedup vs baseline: 1.1859x; 1.1859x over previous
import functools

import jax
import jax.numpy as jnp
from jax import lax
from jax.experimental import pallas as pl
from jax.experimental.pallas import tpu as pltpu
from jax.experimental.pallas import tpu_sc as plsc

H = 512
W = 512
B = 8
N = 4096
HW = H * W

NC = 2
NS = 16
NW = NC * NS
SPW = (B * N) // NW
CHUNKS = SPW // 16

_MESH = plsc.VectorSubcoreMesh(core_axis_name="c", subcore_axis_name="s")


@functools.partial(
    pl.kernel,
    mesh=_MESH,
    out_type=[
        jax.ShapeDtypeStruct((NW, SPW * 2), jnp.int32),
        jax.ShapeDtypeStruct((NW, SPW * 3), jnp.float32),
        jax.ShapeDtypeStruct((NW, SPW * 3), jnp.float32),
    ],
    scratch_types=[
        pltpu.VMEM((SPW,), jnp.int32),
        pltpu.VMEM((SPW * 2,), jnp.int32),
        pltpu.VMEM((SPW * 3,), jnp.int32),
        pltpu.VMEM((SPW * 3,), jnp.float32),
        pltpu.VMEM((SPW * 3,), jnp.float32),
        pltpu.SemaphoreType.DMA,
        pltpu.SemaphoreType.DMA,
    ],
    compiler_params=pltpu.CompilerParams(needs_layout_passes=False),
)
def _sample_gather(idx_hbm, dirs_hbm, orig_hbm,
                   coords_out, dirs_out, orig_out,
                   idx_v, coords_v, fidx_v, dgat_v, ogat_v, dsem, osem):
    wid = lax.axis_index("s") * NC + lax.axis_index("c")
    b = wid // (NW // B)
    pltpu.sync_copy(idx_hbm.at[pl.ds(wid * SPW, SPW)], idx_v)

    iota = lax.iota(jnp.int32, 16)

    def body(j, carry):
        p0 = j * 16
        v = idx_v[pl.ds(p0, 16)]
        qy = 2 * p0 + 2 * iota
        plsc.store_scatter(coords_v, [qy], v >> 9)
        plsc.store_scatter(coords_v, [qy + 1], v & 511)
        q0 = 3 * p0 + 3 * iota
        for c in range(3):
            plsc.store_scatter(fidx_v, [q0 + c], v + ((b * 3 + c) * HW))
        return carry

    lax.fori_loop(0, CHUNKS, body, 0)

    dcp = pltpu.async_copy(dirs_hbm.at[fidx_v], dgat_v, dsem)
    ocp = pltpu.async_copy(orig_hbm.at[fidx_v], ogat_v, osem)
    pltpu.sync_copy(coords_v, coords_out.at[wid])
    dcp.wait()
    pltpu.sync_copy(dgat_v, dirs_out.at[wid])
    ocp.wait()
    pltpu.sync_copy(ogat_v, orig_out.at[wid])


def kernel(n_sample, rays_directions, rays_origins):
    indices = jax.random.randint(jax.random.key(42), (B, N), 0, HW)
    idx_flat = indices.reshape(-1).astype(jnp.int32)
    coords3, dirs3, orig3 = _sample_gather(
        idx_flat,
        rays_directions.reshape(-1),
        rays_origins.reshape(-1),
    )
    sample_coordinates = coords3.reshape(B, N, 2)
    sampled_dirs = dirs3.reshape(B, N, 3)
    sampled_origins = orig3.reshape(B, N, 3)
    indices = indices + (jnp.asarray(n_sample, dtype=indices.dtype) * 0)
    return indices, sample_coordinates, sampled_dirs, sampled_origins


# speedup vs baseline: 1.6067x; 1.3548x over previous
import functools

import jax
import jax.numpy as jnp
from jax import lax
from jax.experimental import pallas as pl
from jax.experimental.pallas import tpu as pltpu
from jax.experimental.pallas import tpu_sc as plsc

H = 512
W = 512
B = 8
N = 4096
HW = H * W

NC = 2
NS = 16
NW = NC * NS
SPW = (B * N) // NW
CHUNKS = SPW // 16

_MESH = plsc.VectorSubcoreMesh(core_axis_name="c", subcore_axis_name="s")


@functools.partial(
    pl.kernel,
    mesh=_MESH,
    out_type=[
        jax.ShapeDtypeStruct((NW, SPW * 2), jnp.int32),
        jax.ShapeDtypeStruct((NW, SPW * 3), jnp.float32),
        jax.ShapeDtypeStruct((NW, SPW * 3), jnp.float32),
    ],
    scratch_types=[
        pltpu.VMEM((SPW,), jnp.int32),
        pltpu.VMEM((SPW * 2,), jnp.int32),
        pltpu.VMEM((SPW * 3,), jnp.int32),
        pltpu.VMEM((SPW * 3,), jnp.float32),
        pltpu.VMEM((SPW * 3,), jnp.float32),
        pltpu.SemaphoreType.DMA,
        pltpu.SemaphoreType.DMA,
    ],
    compiler_params=pltpu.CompilerParams(needs_layout_passes=False),
)
def _sample_gather(idx_hbm, dirs_hbm, orig_hbm,
                   coords_out, dirs_out, orig_out,
                   idx_v, coords_v, fidx_v, dgat_v, ogat_v, dsem, osem):
    wid = lax.axis_index("s") * NC + lax.axis_index("c")
    b = wid // (NW // B)
    pltpu.sync_copy(idx_hbm.at[pl.ds(wid * SPW, SPW)], idx_v)

    iota = lax.iota(jnp.int32, 16)

    def body(j, carry):
        p0 = j * 16
        v = idx_v[pl.ds(p0, 16)]
        y = v >> 9
        x = v & 511
        qy = 2 * p0 + 2 * iota
        plsc.store_scatter(coords_v, [qy], y)
        plsc.store_scatter(coords_v, [qy + 1], x)
        toff = (((y >> 3) << 12) + ((x >> 7) << 10)
                + ((y & 7) << 7) + (x & 127))
        q0 = 3 * p0 + 3 * iota
        for c in range(3):
            plsc.store_scatter(fidx_v, [q0 + c], toff + ((b * 3 + c) * HW))
        return carry

    lax.fori_loop(0, CHUNKS, body, 0)

    dcp = pltpu.async_copy(dirs_hbm.at[fidx_v], dgat_v, dsem)
    ocp = pltpu.async_copy(orig_hbm.at[fidx_v], ogat_v, osem)
    pltpu.sync_copy(coords_v, coords_out.at[wid])
    dcp.wait()
    pltpu.sync_copy(dgat_v, dirs_out.at[wid])
    ocp.wait()
    pltpu.sync_copy(ogat_v, orig_out.at[wid])


def kernel(n_sample, rays_directions, rays_origins):
    indices = jax.random.randint(jax.random.key(42), (B, N), 0, HW)
    idx_flat = indices.reshape(-1).astype(jnp.int32)
    def tiled_flat(a):
        return (a.reshape(B, 3, H // 8, 8, W // 128, 128)
                 .transpose(0, 1, 2, 4, 3, 5)
                 .reshape(-1))

    coords3, dirs3, orig3 = _sample_gather(
        idx_flat,
        tiled_flat(rays_directions),
        tiled_flat(rays_origins),
    )
    sample_coordinates = coords3.reshape(B, N, 2)
    sampled_dirs = dirs3.reshape(B, N, 3)
    sampled_origins = orig3.reshape(B, N, 3)
    indices = indices + (jnp.asarray(n_sample, dtype=indices.dtype) * 0)
    return indices, sample_coordinates, sampled_dirs, sampled_origins
